# Initial kernel scaffold; baseline (speedup 1.0000x reference)
import jax
import jax.numpy as jnp
from jax.experimental import pallas as pl


def kernel(img, inds):
    raise NotImplementedError("write your pallas kernel here")


# speedup vs baseline: 1.2545x; 1.2545x over previous
import functools

import jax
import jax.numpy as jnp
from jax import lax
from jax.experimental import pallas as pl
from jax.experimental.pallas import tpu as pltpu
from jax.experimental.pallas import tpu_sc as plsc

_NC = 2
_NS = 16


def _shuffle(img2, inds, *, C, HW, npairs):
    cpc = C // _NC
    ppt = npairs // _NS
    slw = HW // _NS

    mesh = plsc.VectorSubcoreMesh(core_axis_name="c", subcore_axis_name="s")

    @functools.partial(
        pl.kernel,
        out_type=jax.ShapeDtypeStruct((C, HW), jnp.float32),
        mesh=mesh,
        scratch_types=[
            pltpu.VMEM((ppt,), jnp.int32),
            pltpu.VMEM((ppt,), jnp.int32),
            pltpu.VMEM((ppt,), jnp.float32),
            pltpu.VMEM((ppt,), jnp.float32),
            pltpu.VMEM_SHARED((HW,), jnp.float32),
        ],
    )
    def run(img_hbm, inds_hbm, out_hbm, p_idx, q_idx, vp, vq, rowbuf):
        sc = lax.axis_index("c")
        t = lax.axis_index("s")
        pltpu.sync_copy(inds_hbm.at[pl.ds(t * ppt, ppt)], p_idx)
        pltpu.sync_copy(inds_hbm.at[pl.ds(npairs + t * ppt, ppt)], q_idx)

        def body(k, carry):
            ch = sc * cpc + k
            pltpu.sync_copy(img_hbm.at[ch, pl.ds(t * slw, slw)],
                            rowbuf.at[pl.ds(t * slw, slw)])
            plsc.subcore_barrier()
            pltpu.sync_copy(rowbuf.at[q_idx], vp)
            pltpu.sync_copy(rowbuf.at[p_idx], vq)
            plsc.subcore_barrier()
            pltpu.sync_copy(vp, rowbuf.at[p_idx])
            pltpu.sync_copy(vq, rowbuf.at[q_idx])
            plsc.subcore_barrier()
            pltpu.sync_copy(rowbuf.at[pl.ds(t * slw, slw)],
                            out_hbm.at[ch, pl.ds(t * slw, slw)])
            return carry

        lax.fori_loop(0, cpc, body, 0)

    return run(img2, inds)


def kernel(img, inds):
    C, H, W = img.shape
    HW = H * W
    npairs = inds.shape[0] // 2
    out = _shuffle(img.reshape(C, HW), inds, C=C, HW=HW, npairs=npairs)
    return out.reshape(C, H, W)


# speedup vs baseline: 1.2767x; 1.0177x over previous
import functools

import jax
import jax.numpy as jnp
from jax import lax
from jax.experimental import pallas as pl
from jax.experimental.pallas import tpu as pltpu
from jax.experimental.pallas import tpu_sc as plsc

_NC = 2
_NS = 16


def _shuffle(img2, inds, *, C, HW, npairs):
    cpc = C // _NC
    ppt = npairs // _NS
    slw = HW // _NS

    mesh = plsc.VectorSubcoreMesh(core_axis_name="c", subcore_axis_name="s")

    @functools.partial(
        pl.kernel,
        out_type=jax.ShapeDtypeStruct((C, HW), jnp.float32),
        mesh=mesh,
        scratch_types=[
            pltpu.VMEM((2 * ppt,), jnp.int32),
            pltpu.VMEM((2 * ppt,), jnp.int32),
            pltpu.VMEM((2 * ppt,), jnp.float32),
            pltpu.VMEM_SHARED((HW,), jnp.float32),
        ],
    )
    def run(img_hbm, inds_hbm, out_hbm, g_idx, s_idx, vals, rowbuf):
        sc = lax.axis_index("c")
        t = lax.axis_index("s")
        pltpu.sync_copy(inds_hbm.at[pl.ds(npairs + t * ppt, ppt)],
                        g_idx.at[pl.ds(0, ppt)])
        pltpu.sync_copy(inds_hbm.at[pl.ds(t * ppt, ppt)],
                        g_idx.at[pl.ds(ppt, ppt)])
        pltpu.sync_copy(inds_hbm.at[pl.ds(t * ppt, ppt)],
                        s_idx.at[pl.ds(0, ppt)])
        pltpu.sync_copy(inds_hbm.at[pl.ds(npairs + t * ppt, ppt)],
                        s_idx.at[pl.ds(ppt, ppt)])

        def body(k, carry):
            ch = sc * cpc + k
            pltpu.sync_copy(img_hbm.at[ch, pl.ds(t * slw, slw)],
                            rowbuf.at[pl.ds(t * slw, slw)])
            plsc.subcore_barrier()
            pltpu.sync_copy(rowbuf.at[g_idx], vals)
            pltpu.sync_copy(vals, rowbuf.at[s_idx])
            plsc.subcore_barrier()
            pltpu.sync_copy(rowbuf.at[pl.ds(t * slw, slw)],
                            out_hbm.at[ch, pl.ds(t * slw, slw)])
            return carry

        lax.fori_loop(0, cpc, body, 0)

    return run(img2, inds)


def kernel(img, inds):
    C, H, W = img.shape
    HW = H * W
    npairs = inds.shape[0] // 2
    out = _shuffle(img.reshape(C, HW), inds, C=C, HW=HW, npairs=npairs)
    return out.reshape(C, H, W)


# speedup vs baseline: 1.7543x; 1.3741x over previous
import functools

import jax
import jax.numpy as jnp
from jax import lax
from jax.experimental import pallas as pl
from jax.experimental.pallas import tpu as pltpu
from jax.experimental.pallas import tpu_sc as plsc

_NC = 2
_NS = 16


def _shuffle(img2, inds2, *, C, HW, npairs):
    cpc = C // _NC
    ppt = npairs // _NS
    slw = HW // _NS

    mesh = plsc.VectorSubcoreMesh(core_axis_name="c", subcore_axis_name="s")

    @functools.partial(
        pl.kernel,
        out_type=jax.ShapeDtypeStruct((C, HW), jnp.float32),
        mesh=mesh,
        scratch_types=[
            pltpu.VMEM((2 * ppt,), jnp.int32),
            pltpu.VMEM((2 * ppt,), jnp.int32),
            pltpu.VMEM((2 * ppt,), jnp.float32),
            pltpu.VMEM_SHARED((HW,), jnp.float32),
            pltpu.VMEM_SHARED((HW,), jnp.float32),
            pltpu.SemaphoreType.DMA,
            pltpu.SemaphoreType.DMA,
            pltpu.SemaphoreType.DMA,
            pltpu.SemaphoreType.DMA,
        ],
    )
    def run(img_hbm, inds_hbm, out_hbm, g_idx, s_idx, vals,
            row0, row1, sin0, sin1, sout0, sout1):
        sc = lax.axis_index("c")
        t = lax.axis_index("s")
        ch0 = sc * cpc
        pltpu.sync_copy(inds_hbm.at[pl.ds(npairs + t * ppt, ppt)],
                        g_idx.at[pl.ds(0, ppt)])
        pltpu.sync_copy(inds_hbm.at[pl.ds(t * ppt, ppt)],
                        g_idx.at[pl.ds(ppt, ppt)])
        pltpu.sync_copy(inds_hbm.at[pl.ds(t * ppt, ppt)],
                        s_idx.at[pl.ds(0, ppt)])
        pltpu.sync_copy(inds_hbm.at[pl.ds(npairs + t * ppt, ppt)],
                        s_idx.at[pl.ds(ppt, ppt)])

        sl = pl.ds(t * slw, slw)

        def stage(k, row, sem):
            pltpu.async_copy(img_hbm.at[ch0 + k, sl], row.at[sl], sem)

        def wback(k, row, sem):
            pltpu.async_copy(row.at[sl], out_hbm.at[ch0 + k, sl], sem)

        def drain(row, sem, k):
            pltpu.make_async_copy(row.at[sl], out_hbm.at[ch0 + k, sl],
                                  sem).wait()

        def phase(k, row, sin, other_row, other_sout):
            pltpu.make_async_copy(img_hbm.at[ch0 + k, sl], row.at[sl],
                                  sin).wait()
            plsc.subcore_barrier()
            pltpu.sync_copy(row.at[g_idx], vals)

            @pl.when(k >= 1)
            def _():
                drain(other_row, other_sout, k - 1)

            @pl.when(k + 1 < cpc)
            def _():
                stage(k + 1, other_row, sin1 if row is row0 else sin0)

            pltpu.sync_copy(vals, row.at[s_idx])
            plsc.subcore_barrier()
            wback(k, row, sout0 if row is row0 else sout1)

        stage(0, row0, sin0)

        def body(k2, carry):
            phase(2 * k2, row0, sin0, row1, sout1)
            phase(2 * k2 + 1, row1, sin1, row0, sout0)
            return carry

        lax.fori_loop(0, cpc // 2, body, 0)
        drain(row1, sout1, cpc - 1)

    return run(img2, inds2)


def kernel(img, inds):
    C, H, W = img.shape
    HW = H * W
    npairs = inds.shape[0] // 2
    r, c = inds // W, inds % W
    phys = (((r >> 3) * (W // 128) + (c >> 7)) << 10) | ((r & 7) << 7) | (c & 127)
    flat = (img.reshape(C, H // 8, 8, W // 128, 128)
            .transpose(0, 1, 3, 2, 4)
            .reshape(C, HW))
    out = _shuffle(flat, phys, C=C, HW=HW, npairs=npairs)
    return (out.reshape(C, H // 8, W // 128, 8, 128)
            .transpose(0, 1, 3, 2, 4)
            .reshape(C, H, W))


# speedup vs baseline: 1.7686x; 1.0081x over previous
import functools

import jax
import jax.numpy as jnp
from jax import lax
from jax.experimental import pallas as pl
from jax.experimental.pallas import tpu as pltpu
from jax.experimental.pallas import tpu_sc as plsc

_NC = 2
_NS = 16


def _shuffle(img2, inds2, *, C, HW, npairs):
    cpc = C // _NC
    ppt = npairs // _NS
    hpt = ppt // 2
    slw = HW // _NS

    mesh = plsc.VectorSubcoreMesh(core_axis_name="c", subcore_axis_name="s")

    @functools.partial(
        pl.kernel,
        out_type=jax.ShapeDtypeStruct((C, HW), jnp.float32),
        mesh=mesh,
        scratch_types=[
            pltpu.VMEM((ppt,), jnp.int32),
            pltpu.VMEM((ppt,), jnp.int32),
            pltpu.VMEM((ppt,), jnp.int32),
            pltpu.VMEM((ppt,), jnp.int32),
            pltpu.VMEM((ppt,), jnp.float32),
            pltpu.VMEM((ppt,), jnp.float32),
            pltpu.VMEM_SHARED((HW,), jnp.float32),
            pltpu.VMEM_SHARED((HW,), jnp.float32),
            pltpu.SemaphoreType.DMA,
            pltpu.SemaphoreType.DMA,
            pltpu.SemaphoreType.DMA,
            pltpu.SemaphoreType.DMA,
            pltpu.SemaphoreType.DMA,
            pltpu.SemaphoreType.DMA,
            pltpu.SemaphoreType.DMA,
        ],
    )
    def run(img_hbm, inds_hbm, out_hbm, g0i, g1i, s0i, s1i, v0, v1,
            row0, row1, sin0, sin1, sout0, sout1, sg0, sg1, ss):
        sc = lax.axis_index("c")
        t = lax.axis_index("s")
        ch0 = sc * cpc
        for gi, si, c in ((g0i, s0i, 0), (g1i, s1i, 1)):
            pltpu.sync_copy(inds_hbm.at[pl.ds(npairs + t * ppt + c * hpt, hpt)],
                            gi.at[pl.ds(0, hpt)])
            pltpu.sync_copy(inds_hbm.at[pl.ds(t * ppt + c * hpt, hpt)],
                            gi.at[pl.ds(hpt, hpt)])
            pltpu.sync_copy(inds_hbm.at[pl.ds(t * ppt + c * hpt, hpt)],
                            si.at[pl.ds(0, hpt)])
            pltpu.sync_copy(inds_hbm.at[pl.ds(npairs + t * ppt + c * hpt, hpt)],
                            si.at[pl.ds(hpt, hpt)])

        sl = pl.ds(t * slw, slw)

        def stage(k, row, sem):
            pltpu.async_copy(img_hbm.at[ch0 + k, sl], row.at[sl], sem)

        def wback(k, row, sem):
            pltpu.async_copy(row.at[sl], out_hbm.at[ch0 + k, sl], sem)

        def drain(row, sem, k):
            pltpu.make_async_copy(row.at[sl], out_hbm.at[ch0 + k, sl],
                                  sem).wait()

        def phase(k, row, sin, other_row, other_sout):
            pltpu.make_async_copy(img_hbm.at[ch0 + k, sl], row.at[sl],
                                  sin).wait()
            plsc.subcore_barrier()
            d_g0 = pltpu.async_copy(row.at[g0i], v0, sg0)
            d_g1 = pltpu.async_copy(row.at[g1i], v1, sg1)
            d_g0.wait()
            d_s0 = pltpu.async_copy(v0, row.at[s0i], ss)

            @pl.when(k >= 1)
            def _():
                drain(other_row, other_sout, k - 1)

            @pl.when(k + 1 < cpc)
            def _():
                stage(k + 1, other_row, sin1 if row is row0 else sin0)

            d_g1.wait()
            d_s1 = pltpu.async_copy(v1, row.at[s1i], ss)
            d_s0.wait()
            d_s1.wait()
            plsc.subcore_barrier()
            wback(k, row, sout0 if row is row0 else sout1)

        stage(0, row0, sin0)

        def body(k2, carry):
            phase(2 * k2, row0, sin0, row1, sout1)
            phase(2 * k2 + 1, row1, sin1, row0, sout0)
            return carry

        lax.fori_loop(0, cpc // 2, body, 0)
        drain(row1, sout1, cpc - 1)

    return run(img2, inds2)


def kernel(img, inds):
    C, H, W = img.shape
    HW = H * W
    npairs = inds.shape[0] // 2
    r, c = inds // W, inds % W
    phys = (((r >> 3) * (W // 128) + (c >> 7)) << 10) | ((r & 7) << 7) | (c & 127)
    flat = (img.reshape(C, H // 8, 8, W // 128, 128)
            .transpose(0, 1, 3, 2, 4)
            .reshape(C, HW))
    out = _shuffle(flat, phys, C=C, HW=HW, npairs=npairs)
    return (out.reshape(C, H // 8, W // 128, 8, 128)
            .transpose(0, 1, 3, 2, 4)
            .reshape(C, H, W))
